# baseline (speedup 1.0000x reference)
import functools

import jax
import jax.numpy as jnp
from jax import lax
from jax.experimental import pallas as pl
from jax.experimental.pallas import tpu as pltpu
from jax.experimental.pallas import tpu_sc as plsc

NR_CATE = 26
BATCH = 16384
NR_FEAT = 128

NC = 2
NS = 16
L = 16
NW = NC * NS
BPW = BATCH // NW
CHUNK = 128
NCHUNK = BPW // CHUNK

_mesh = plsc.VectorSubcoreMesh(core_axis_name="c", subcore_axis_name="s")


@functools.partial(
    pl.kernel,
    mesh=_mesh,
    out_type=jax.ShapeDtypeStruct((BATCH, NR_FEAT), jnp.float32),
    compiler_params=pltpu.CompilerParams(use_tc_tiling_on_sc=True),
    scratch_types=[
        pltpu.VMEM((BPW,), jnp.int32),
        pltpu.VMEM((NCHUNK, CHUNK), jnp.int32),
        pltpu.VMEM((BPW, NR_FEAT), jnp.float32),
        pltpu.SemaphoreType.DMA,
        pltpu.SemaphoreType.DMA,
    ],
)
def _maskout_sc(x_hbm, label_hbm, out_hbm, lab_v, idx_v, rows_v, sem, out_sem):
    wid = lax.axis_index("s") * NC + lax.axis_index("c")
    base = wid * BPW

    pltpu.sync_copy(label_hbm.at[pl.ds(base, BPW)], lab_v)

    iota = lax.iota(jnp.int32, L)
    for j in range(BPW // L):
        lab16 = lab_v[pl.ds(j * L, L)]
        idx16 = lab16 * BATCH + iota + (base + j * L)
        idx_v[j * L // CHUNK, pl.ds(j * L % CHUNK, L)] = idx16

    gathers = [
        pltpu.async_copy(
            x_hbm.at[idx_v.at[k]],
            rows_v.at[pl.ds(k * CHUNK, CHUNK)],
            sem,
        )
        for k in range(NCHUNK)
    ]
    writes = []
    for k in range(NCHUNK):
        gathers[k].wait()
        writes.append(
            pltpu.async_copy(
                rows_v.at[pl.ds(k * CHUNK, CHUNK)],
                out_hbm.at[pl.ds(base + k * CHUNK, CHUNK)],
                out_sem,
            )
        )
    for cp in writes:
        cp.wait()


def kernel(x, label):
    x_flat = jnp.transpose(x, (1, 0, 2)).reshape(NR_CATE * BATCH, NR_FEAT)
    return _maskout_sc(x_flat, label)


# speedup vs baseline: 1.0117x; 1.0117x over previous
import functools

import jax
import jax.numpy as jnp
from jax import lax
from jax.experimental import pallas as pl
from jax.experimental.pallas import tpu as pltpu
from jax.experimental.pallas import tpu_sc as plsc

NR_CATE = 26
BATCH = 16384
NR_FEAT = 128

NC = 2
NS = 16
L = 16
NW = NC * NS
BPW = BATCH // NW
CHUNK = 128
NCHUNK = BPW // CHUNK

_mesh = plsc.VectorSubcoreMesh(core_axis_name="c", subcore_axis_name="s")


@functools.partial(
    pl.kernel,
    mesh=_mesh,
    out_type=jax.ShapeDtypeStruct((BATCH, NR_FEAT), jnp.float32),
    compiler_params=pltpu.CompilerParams(use_tc_tiling_on_sc=True),
    scratch_types=[
        pltpu.VMEM((BPW,), jnp.int32),
        pltpu.VMEM((NCHUNK, CHUNK), jnp.int32),
        pltpu.VMEM((BPW, NR_FEAT), jnp.float32),
        pltpu.SemaphoreType.DMA,
    ],
)
def _maskout_sc(x_hbm, label_hbm, out_hbm, lab_v, idx_v, rows_v, sem):
    wid = lax.axis_index("s") * NC + lax.axis_index("c")
    base = wid * BPW

    pltpu.sync_copy(label_hbm.at[pl.ds(base, BPW)], lab_v)

    iota = lax.iota(jnp.int32, L)
    gathers = []
    for k in range(NCHUNK):
        for j in range(k * (CHUNK // L), (k + 1) * (CHUNK // L)):
            lab16 = lab_v[pl.ds(j * L, L)]
            idx16 = lab16 * BATCH + iota + (base + j * L)
            idx_v[k, pl.ds(j * L % CHUNK, L)] = idx16
        gathers.append(
            pltpu.async_copy(
                x_hbm.at[idx_v.at[k]],
                rows_v.at[pl.ds(k * CHUNK, CHUNK)],
                sem,
            )
        )
    for cp in gathers:
        cp.wait()

    pltpu.sync_copy(rows_v, out_hbm.at[pl.ds(base, BPW)])


def kernel(x, label):
    x_flat = jnp.transpose(x, (1, 0, 2)).reshape(NR_CATE * BATCH, NR_FEAT)
    return _maskout_sc(x_flat, label)
